# baseline (speedup 1.0000x reference)
import jax
import jax.numpy as jnp
from jax import lax
from jax.experimental import pallas as pl
from jax.experimental.pallas import tpu as pltpu
from jax.experimental.pallas import tpu_sc as plsc

N = 10000
E = 320000
DIM = 128
HEADS = 8
DH = 16
EPS = 1e-6

NC = 2
NS = 16
NW = NC * NS
EPW = E // NW
C1 = 50
NCH1 = EPW // C1
C2 = 100
NCH2 = EPW // C2
NP = 10112
RPT = NP // NS

RB = 1000
GRID = N // RB

W16 = 16


def _lrelu(v):
    return jnp.maximum(v, 0.2 * v)


def _tca_body(x_ref, w_ref, as_w_ref, ad_w_ref, h_ref, as_ref, ad_ref,
              maxs_ref, maxd_ref):
    i = pl.program_id(0)
    h = jnp.dot(x_ref[...], w_ref[...], preferred_element_type=jnp.float32)
    h_ref[...] = h
    a_s = jnp.dot(h, as_w_ref[...], preferred_element_type=jnp.float32)
    a_d = jnp.dot(h, ad_w_ref[...], preferred_element_type=jnp.float32)
    as_ref[...] = a_s
    ad_ref[...] = a_d
    bs = jnp.max(a_s, axis=0, keepdims=True)
    bd = jnp.max(a_d, axis=0, keepdims=True)

    @pl.when(i == 0)
    def _():
        maxs_ref[...] = bs
        maxd_ref[...] = bd

    @pl.when(i > 0)
    def _():
        maxs_ref[...] = jnp.maximum(maxs_ref[...], bs)
        maxd_ref[...] = jnp.maximum(maxd_ref[...], bd)


def _tca(x, w, as_w, ad_w):
    return pl.pallas_call(
        _tca_body,
        grid=(GRID,),
        in_specs=[
            pl.BlockSpec((RB, DIM), lambda i: (i, 0)),
            pl.BlockSpec((DIM, DIM), lambda i: (0, 0)),
            pl.BlockSpec((DIM, W16), lambda i: (0, 0)),
            pl.BlockSpec((DIM, W16), lambda i: (0, 0)),
        ],
        out_specs=[
            pl.BlockSpec((RB, DIM), lambda i: (i, 0)),
            pl.BlockSpec((RB, W16), lambda i: (i, 0)),
            pl.BlockSpec((RB, W16), lambda i: (i, 0)),
            pl.BlockSpec((1, W16), lambda i: (0, 0)),
            pl.BlockSpec((1, W16), lambda i: (0, 0)),
        ],
        out_shape=[
            jax.ShapeDtypeStruct((N, DIM), jnp.float32),
            jax.ShapeDtypeStruct((N, W16), jnp.float32),
            jax.ShapeDtypeStruct((N, W16), jnp.float32),
            jax.ShapeDtypeStruct((1, W16), jnp.float32),
            jax.ShapeDtypeStruct((1, W16), jnp.float32),
        ],
    )(x, w, as_w, ad_w)


def _sc0_body(src_hbm, z8_hbm, ones_hbm, deg_out,
              srcb, onesv, ssem, deg_sh):
    cid = lax.axis_index("c")
    sid = lax.axis_index("s")
    wid = sid * NC + cid

    r0 = sid * RPT
    pltpu.sync_copy(z8_hbm, deg_sh.at[pl.ds(r0, RPT)])
    pltpu.sync_copy(src_hbm.at[wid], srcb)
    pltpu.sync_copy(ones_hbm, onesv)
    plsc.subcore_barrier()

    def fire(i, _):
        pltpu.async_copy(onesv, deg_sh.at[srcb.at[i]], ssem, add=True)
        return ()

    def drain(i, _):
        pltpu.make_async_copy(onesv, deg_sh.at[srcb.at[i]], ssem).wait()
        return ()

    lax.fori_loop(0, NCH1, fire, (), unroll=False)
    lax.fori_loop(0, NCH1, drain, (), unroll=False)
    plsc.subcore_barrier()
    pltpu.sync_copy(deg_sh.at[pl.ds(r0, RPT)],
                    deg_out.at[cid, pl.ds(r0, RPT)])


def _sc0(src_r, z8, ones1):
    f = pl.kernel(
        _sc0_body,
        out_type=[jax.ShapeDtypeStruct((NC, NP, HEADS), jnp.float32)],
        mesh=plsc.VectorSubcoreMesh(core_axis_name="c", subcore_axis_name="s",
                                    num_cores=NC, num_subcores=NS),
        compiler_params=pltpu.CompilerParams(use_tc_tiling_on_sc=False),
        scratch_types=[
            pltpu.VMEM((NCH1, C1), jnp.int32),
            pltpu.VMEM((C1, HEADS), jnp.float32),
            pltpu.SemaphoreType.DMA,
            pltpu.VMEM_SHARED((NP, HEADS), jnp.float32),
        ],
    )
    return f(src_r, z8, ones1)[0]


def _sc1_body(src_hbm, dst_hbm, as_hbm, ad_hbm, h_hbm, bnd_hbm, z128_hbm,
              z16_hbm,
              numer_out, denom_out,
              srcb, dstb, asg0, asg1, adg0, adg1, exv0, exv1, hv0, hv1,
              bndv, gsem0, gsem1, ssem0, ssem1,
              numer_sh, denom_sh):
    cid = lax.axis_index("c")
    sid = lax.axis_index("s")
    wid = sid * NC + cid
    asg = (asg0, asg1)
    adg = (adg0, adg1)
    exv = (exv0, exv1)
    hv = (hv0, hv1)
    gsem = (gsem0, gsem1)
    ssem = (ssem0, ssem1)

    r0 = sid * RPT
    pltpu.sync_copy(z128_hbm, numer_sh.at[pl.ds(r0, RPT)])
    pltpu.sync_copy(z16_hbm, denom_sh.at[pl.ds(r0, RPT)])
    pltpu.sync_copy(src_hbm.at[wid], srcb)
    pltpu.sync_copy(dst_hbm.at[wid], dstb)
    pltpu.sync_copy(bnd_hbm, bndv)
    pltpu.sync_copy(z16_hbm.at[pl.ds(0, C1)], exv1)
    pltpu.sync_copy(z128_hbm.at[pl.ds(0, C1)], hv1)
    plsc.subcore_barrier()

    bnd = bndv[...]

    pltpu.async_copy(as_hbm.at[srcb.at[0]], asg0, gsem0)
    pltpu.async_copy(ad_hbm.at[dstb.at[0]], adg0, gsem0)
    pltpu.async_copy(h_hbm.at[srcb.at[0]], hv0, gsem0)
    pltpu.async_copy(exv1, denom_sh.at[dstb.at[0]], ssem1, add=True)
    pltpu.async_copy(hv1, numer_sh.at[dstb.at[0]], ssem1, add=True)

    def step(j, _):
        for b in range(2):
            i = 2 * j + b
            nb = 1 - b
            si = srcb.at[i]
            di = dstb.at[i]
            inx = jnp.minimum(i + 1, NCH1 - 1)
            sin = srcb.at[inx]
            din = dstb.at[inx]
            pltpu.async_copy(as_hbm.at[sin], asg[nb], gsem[nb])
            pltpu.async_copy(ad_hbm.at[din], adg[nb], gsem[nb])
            pltpu.make_async_copy(as_hbm.at[si], asg[b], gsem[b]).wait()
            pltpu.make_async_copy(ad_hbm.at[di], adg[b], gsem[b]).wait()
            pltpu.make_async_copy(h_hbm.at[si], hv[b], gsem[b]).wait()
            for r in range(C1):
                s = _lrelu(asg[b][r, :] + adg[b][r, :])
                exv[b][r, :] = jnp.exp(s - bnd)
            pltpu.async_copy(exv[b], denom_sh.at[di], ssem[b], add=True)
            pltpu.make_async_copy(exv[nb], denom_sh.at[di], ssem[nb]).wait()
            pltpu.make_async_copy(hv[nb], numer_sh.at[di], ssem[nb]).wait()
            pltpu.async_copy(h_hbm.at[sin], hv[nb], gsem[nb])
            for e in range(C1):
                exrow = exv[b][e, :]
                for k in range(HEADS):
                    hv[b][e, pl.ds(k * DH, DH)] = (
                        hv[b][e, pl.ds(k * DH, DH)] * exrow[k])
            pltpu.async_copy(hv[b], numer_sh.at[di], ssem[b], add=True)
        return ()

    lax.fori_loop(0, NCH1 // 2, step, (), unroll=False)
    pltpu.make_async_copy(as_hbm.at[srcb.at[0]], asg0, gsem0).wait()
    pltpu.make_async_copy(ad_hbm.at[dstb.at[0]], adg0, gsem0).wait()
    pltpu.make_async_copy(h_hbm.at[srcb.at[0]], hv0, gsem0).wait()
    pltpu.make_async_copy(exv1, denom_sh.at[dstb.at[0]], ssem1).wait()
    pltpu.make_async_copy(hv1, numer_sh.at[dstb.at[0]], ssem1).wait()
    plsc.subcore_barrier()
    pltpu.sync_copy(numer_sh.at[pl.ds(r0, RPT)],
                    numer_out.at[cid, pl.ds(r0, RPT)])
    pltpu.sync_copy(denom_sh.at[pl.ds(r0, RPT)],
                    denom_out.at[cid, pl.ds(r0, RPT)])


def _sc1(src_r, dst_r, as_tab, ad_tab, h, bnd16, z128, z16):
    f = pl.kernel(
        _sc1_body,
        out_type=[
            jax.ShapeDtypeStruct((NC, NP, DIM), jnp.float32),
            jax.ShapeDtypeStruct((NC, NP, W16), jnp.float32),
        ],
        mesh=plsc.VectorSubcoreMesh(core_axis_name="c", subcore_axis_name="s",
                                    num_cores=NC, num_subcores=NS),
        compiler_params=pltpu.CompilerParams(use_tc_tiling_on_sc=False),
        scratch_types=[
            pltpu.VMEM((NCH1, C1), jnp.int32),
            pltpu.VMEM((NCH1, C1), jnp.int32),
            pltpu.VMEM((C1, W16), jnp.float32),
            pltpu.VMEM((C1, W16), jnp.float32),
            pltpu.VMEM((C1, W16), jnp.float32),
            pltpu.VMEM((C1, W16), jnp.float32),
            pltpu.VMEM((C1, W16), jnp.float32),
            pltpu.VMEM((C1, W16), jnp.float32),
            pltpu.VMEM((C1, DIM), jnp.float32),
            pltpu.VMEM((C1, DIM), jnp.float32),
            pltpu.VMEM((W16,), jnp.float32),
            pltpu.SemaphoreType.DMA,
            pltpu.SemaphoreType.DMA,
            pltpu.SemaphoreType.DMA,
            pltpu.SemaphoreType.DMA,
            pltpu.VMEM_SHARED((NP, DIM), jnp.float32),
            pltpu.VMEM_SHARED((NP, W16), jnp.float32),
        ],
    )
    return f(src_r, dst_r, as_tab, ad_tab, h, bnd16, z128, z16)


def _sc2_body(src_hbm, dst_hbm, tab_hbm, z16_hbm, acc_out,
              srcb, dstb, tv0, tv1, tv2, tv3,
              gsem0, gsem1, gsem2, gsem3, ssem0, ssem1, ssem2, ssem3,
              acc_sh):
    cid = lax.axis_index("c")
    sid = lax.axis_index("s")
    wid = sid * NC + cid
    tv = (tv0, tv1, tv2, tv3)
    gsem = (gsem0, gsem1, gsem2, gsem3)
    ssem = (ssem0, ssem1, ssem2, ssem3)

    r0 = sid * RPT
    pltpu.sync_copy(z16_hbm, acc_sh.at[pl.ds(r0, RPT)])
    pltpu.sync_copy(src_hbm.at[wid], srcb)
    pltpu.sync_copy(dst_hbm.at[wid], dstb)
    pltpu.sync_copy(z16_hbm.at[pl.ds(0, C2)], tv2)
    pltpu.sync_copy(z16_hbm.at[pl.ds(0, C2)], tv3)
    plsc.subcore_barrier()

    pltpu.async_copy(tab_hbm.at[dstb.at[0]], tv0, gsem0)
    pltpu.async_copy(tab_hbm.at[dstb.at[1]], tv1, gsem1)
    pltpu.async_copy(tv2, acc_sh.at[srcb.at[0]], ssem2, add=True)
    pltpu.async_copy(tv3, acc_sh.at[srcb.at[0]], ssem3, add=True)

    def step(j, _):
        for b in range(4):
            i = 4 * j + b
            pb = (b + 2) % 4
            si = srcb.at[i]
            inx = jnp.minimum(i + 2, NCH2 - 1)
            pltpu.make_async_copy(tv[pb], acc_sh.at[si], ssem[pb]).wait()
            pltpu.async_copy(tab_hbm.at[dstb.at[inx]], tv[pb], gsem[pb])
            pltpu.make_async_copy(tab_hbm.at[dstb.at[i]], tv[b], gsem[b]).wait()
            pltpu.async_copy(tv[b], acc_sh.at[si], ssem[b], add=True)
        return ()

    lax.fori_loop(0, NCH2 // 4, step, (), unroll=False)
    pltpu.make_async_copy(tab_hbm.at[dstb.at[0]], tv0, gsem0).wait()
    pltpu.make_async_copy(tab_hbm.at[dstb.at[0]], tv1, gsem1).wait()
    pltpu.make_async_copy(tv2, acc_sh.at[srcb.at[0]], ssem2).wait()
    pltpu.make_async_copy(tv3, acc_sh.at[srcb.at[0]], ssem3).wait()
    plsc.subcore_barrier()
    pltpu.sync_copy(acc_sh.at[pl.ds(r0, RPT)],
                    acc_out.at[cid, pl.ds(r0, RPT)])


def _sc2(src_r, dst_r, tab, z16):
    f = pl.kernel(
        _sc2_body,
        out_type=[jax.ShapeDtypeStruct((NC, NP, W16), jnp.float32)],
        mesh=plsc.VectorSubcoreMesh(core_axis_name="c", subcore_axis_name="s",
                                    num_cores=NC, num_subcores=NS),
        compiler_params=pltpu.CompilerParams(use_tc_tiling_on_sc=False),
        scratch_types=[
            pltpu.VMEM((NCH2, C2), jnp.int32),
            pltpu.VMEM((NCH2, C2), jnp.int32),
            pltpu.VMEM((C2, W16), jnp.float32),
            pltpu.VMEM((C2, W16), jnp.float32),
            pltpu.VMEM((C2, W16), jnp.float32),
            pltpu.VMEM((C2, W16), jnp.float32),
            pltpu.SemaphoreType.DMA,
            pltpu.SemaphoreType.DMA,
            pltpu.SemaphoreType.DMA,
            pltpu.SemaphoreType.DMA,
            pltpu.SemaphoreType.DMA,
            pltpu.SemaphoreType.DMA,
            pltpu.SemaphoreType.DMA,
            pltpu.SemaphoreType.DMA,
            pltpu.VMEM_SHARED((NP, W16), jnp.float32),
        ],
    )
    return f(src_r, dst_r, tab, z16)[0]


def _tcb_body(nump_ref, denp_ref, degp_ref, h_ref, as_ref, ad_ref, bnd_ref,
              r_ref, bias_ref, xh_ref, t1_ref, dis_ref, m_ref, msq_ref):
    ss = as_ref[:, :HEADS] + ad_ref[:, :HEADS]
    ex_self = jnp.exp(_lrelu(ss) - bnd_ref[:, :HEADS])
    den8 = denp_ref[0][:, :HEADS] + denp_ref[1][:, :HEADS] + ex_self + 1e-16
    denx = jnp.dot(den8, r_ref[...], preferred_element_type=jnp.float32)
    numx = (nump_ref[0] + nump_ref[1]
            + h_ref[...] * jnp.dot(ex_self, r_ref[...],
                                   preferred_element_type=jnp.float32))
    xh = numx / denx + bias_ref[...]
    xh_ref[...] = xh
    m = jnp.mean(xh, axis=1, keepdims=True)
    msq = jnp.mean(xh * xh, axis=1, keepdims=True)
    deg = degp_ref[0][:, 0:1] + degp_ref[1][:, 0:1]
    dis = lax.rsqrt(deg + EPS)
    dis_ref[...] = dis
    m_ref[...] = m
    msq_ref[...] = msq
    col = lax.broadcasted_iota(jnp.int32, (RB, W16), 1)
    t1a = jnp.broadcast_to(dis * m, (RB, W16))
    t1b = jnp.broadcast_to(dis * msq, (RB, W16))
    t1_ref[...] = jnp.where(col == 0, t1a, jnp.where(col == 1, t1b, 0.0))


def _tcb(nump, denp, degp, h, as_tab, ad_tab, bnd8, r_exp, bias_row):
    return pl.pallas_call(
        _tcb_body,
        grid=(GRID,),
        in_specs=[
            pl.BlockSpec((NC, RB, DIM), lambda i: (0, i, 0)),
            pl.BlockSpec((NC, RB, W16), lambda i: (0, i, 0)),
            pl.BlockSpec((NC, RB, HEADS), lambda i: (0, i, 0)),
            pl.BlockSpec((RB, DIM), lambda i: (i, 0)),
            pl.BlockSpec((RB, W16), lambda i: (i, 0)),
            pl.BlockSpec((RB, W16), lambda i: (i, 0)),
            pl.BlockSpec((1, W16), lambda i: (0, 0)),
            pl.BlockSpec((HEADS, DIM), lambda i: (0, 0)),
            pl.BlockSpec((1, DIM), lambda i: (0, 0)),
        ],
        out_specs=[
            pl.BlockSpec((RB, DIM), lambda i: (i, 0)),
            pl.BlockSpec((RB, W16), lambda i: (i, 0)),
            pl.BlockSpec((RB, 1), lambda i: (i, 0)),
            pl.BlockSpec((RB, 1), lambda i: (i, 0)),
            pl.BlockSpec((RB, 1), lambda i: (i, 0)),
        ],
        out_shape=[
            jax.ShapeDtypeStruct((N, DIM), jnp.float32),
            jax.ShapeDtypeStruct((N, W16), jnp.float32),
            jax.ShapeDtypeStruct((N, 1), jnp.float32),
            jax.ShapeDtypeStruct((N, 1), jnp.float32),
            jax.ShapeDtypeStruct((N, 1), jnp.float32),
        ],
    )(nump, denp, degp, h, as_tab, ad_tab, bnd8, r_exp, bias_row)


def _tcb2_body(acc1_ref, dis_ref, m_ref, t2_ref, mean_ref, bv_ref):
    a1 = acc1_ref[0] + acc1_ref[1]
    dis = dis_ref[...]
    mean = dis * a1[:, 0:1]
    bv = dis * a1[:, 1:2]
    mean_ref[...] = mean
    bv_ref[...] = bv
    t2a = dis * (2.0 * mean * m_ref[...] - mean * mean)
    col = lax.broadcasted_iota(jnp.int32, (RB, W16), 1)
    t2_ref[...] = jnp.where(col == 0, jnp.broadcast_to(t2a, (RB, W16)), 0.0)


def _tcb2(acc1, dis, m):
    return pl.pallas_call(
        _tcb2_body,
        grid=(GRID,),
        in_specs=[
            pl.BlockSpec((NC, RB, W16), lambda i: (0, i, 0)),
            pl.BlockSpec((RB, 1), lambda i: (i, 0)),
            pl.BlockSpec((RB, 1), lambda i: (i, 0)),
        ],
        out_specs=[
            pl.BlockSpec((RB, W16), lambda i: (i, 0)),
            pl.BlockSpec((RB, 1), lambda i: (i, 0)),
            pl.BlockSpec((RB, 1), lambda i: (i, 0)),
        ],
        out_shape=[
            jax.ShapeDtypeStruct((N, W16), jnp.float32),
            jax.ShapeDtypeStruct((N, 1), jnp.float32),
            jax.ShapeDtypeStruct((N, 1), jnp.float32),
        ],
    )(acc1, dis, m)


def _tcc_body(xh_ref, x_ref, mean_ref, bv_ref, dis_ref, acc2_ref,
              lsc_ref, lbi_ref, lw_ref, lb_ref, w1_ref, b1_ref, w2_ref,
              b2_ref, out_ref):
    a2 = acc2_ref[0][:, 0:1] + acc2_ref[1][:, 0:1]
    var = bv_ref[...] - dis_ref[...] * a2
    xh = xh_ref[...]
    xn = (xh - mean_ref[...]) / jnp.sqrt(var + EPS) * lsc_ref[...] + lbi_ref[...]
    out = xn + x_ref[...]
    mu = jnp.mean(out, axis=1, keepdims=True)
    d = out - mu
    vv = jnp.mean(d * d, axis=1, keepdims=True)
    o1 = d / jnp.sqrt(vv + 1e-5) * lw_ref[...] + lb_ref[...]
    hpre = jnp.dot(o1, w1_ref[...], preferred_element_type=jnp.float32) + b1_ref[...]
    hh = 0.5 * hpre * (1.0 + lax.erf(hpre * 0.7071067811865476))
    o2 = jnp.dot(hh, w2_ref[...], preferred_element_type=jnp.float32) + b2_ref[...] + o1
    nrm = jnp.sqrt(jnp.sum(o2 * o2, axis=1, keepdims=True))
    out_ref[...] = o2 / jnp.maximum(nrm, 1e-12)


def _tcc(xh, x, mean, bv, dis, acc2, lsc, lbi, lw, lb, w1, b1, w2, b2):
    return pl.pallas_call(
        _tcc_body,
        grid=(GRID,),
        in_specs=[
            pl.BlockSpec((RB, DIM), lambda i: (i, 0)),
            pl.BlockSpec((RB, DIM), lambda i: (i, 0)),
            pl.BlockSpec((RB, 1), lambda i: (i, 0)),
            pl.BlockSpec((RB, 1), lambda i: (i, 0)),
            pl.BlockSpec((RB, 1), lambda i: (i, 0)),
            pl.BlockSpec((NC, RB, W16), lambda i: (0, i, 0)),
            pl.BlockSpec((1, DIM), lambda i: (0, 0)),
            pl.BlockSpec((1, DIM), lambda i: (0, 0)),
            pl.BlockSpec((1, DIM), lambda i: (0, 0)),
            pl.BlockSpec((1, DIM), lambda i: (0, 0)),
            pl.BlockSpec((DIM, 2 * DIM), lambda i: (0, 0)),
            pl.BlockSpec((1, 2 * DIM), lambda i: (0, 0)),
            pl.BlockSpec((2 * DIM, DIM), lambda i: (0, 0)),
            pl.BlockSpec((1, DIM), lambda i: (0, 0)),
        ],
        out_specs=pl.BlockSpec((RB, DIM), lambda i: (i, 0)),
        out_shape=jax.ShapeDtypeStruct((N, DIM), jnp.float32),
    )(xh, x, mean, bv, dis, acc2, lsc, lbi, lw, lb, w1, b1, w2, b2)


def kernel(x, edge_index, W_gat, att_src, att_dst, bias_gat, lap_scale,
           lap_bias, ln2_w, ln2_b, W1, b1, W2, b2):
    src = edge_index[0].astype(jnp.int32)
    dst = edge_index[1].astype(jnp.int32)
    src1 = src.reshape(NW, NCH1, C1)
    dst1 = dst.reshape(NW, NCH1, C1)
    src2 = src.reshape(NW, NCH2, C2)
    dst2 = dst.reshape(NW, NCH2, C2)

    lanes = jnp.arange(DIM, dtype=jnp.int32)
    hsel = (lanes[:, None] // DH) == (jnp.arange(W16, dtype=jnp.int32) % HEADS)[None, :]
    as_w = jnp.where(hsel, att_src.reshape(DIM)[:, None], 0.0)
    ad_w = jnp.where(hsel, att_dst.reshape(DIM)[:, None], 0.0)
    sel8 = (lanes[:, None] // DH) == jnp.arange(HEADS, dtype=jnp.int32)[None, :]
    r_exp = sel8.T.astype(jnp.float32)

    h, as_tab, ad_tab, maxs, maxd = _tca(x, W_gat, as_w, ad_w)
    bnd = _lrelu(maxs + maxd)
    bnd16 = bnd.reshape(W16)

    z128 = jnp.zeros((RPT, DIM), jnp.float32)
    z16 = jnp.zeros((RPT, W16), jnp.float32)
    z8 = jnp.zeros((RPT, HEADS), jnp.float32)
    ones1 = jnp.ones((C1, HEADS), jnp.float32)

    degp = _sc0(src1, z8, ones1)
    nump, denp = _sc1(src1, dst1, as_tab, ad_tab, h, bnd16, z128, z16)

    xh, t1, dis, m, msq = _tcb(nump, denp, degp, h, as_tab, ad_tab, bnd,
                               r_exp, bias_gat.reshape(1, DIM))

    acc1 = _sc2(src2, dst2, t1, z16)
    t2, mean, bv = _tcb2(acc1, dis, m)
    acc2 = _sc2(src2, dst2, t2, z16)

    return _tcc(xh, x, mean, bv, dis, acc2,
                lap_scale.reshape(1, DIM), lap_bias.reshape(1, DIM),
                ln2_w.reshape(1, DIM), ln2_b.reshape(1, DIM),
                W1, b1.reshape(1, 2 * DIM), W2, b2.reshape(1, DIM))


# speedup vs baseline: 1.0698x; 1.0698x over previous
import jax
import jax.numpy as jnp
from jax import lax
from jax.experimental import pallas as pl
from jax.experimental.pallas import tpu as pltpu
from jax.experimental.pallas import tpu_sc as plsc

N = 10000
E = 320000
DIM = 128
HEADS = 8
DH = 16
EPS = 1e-6

NC = 2
NS = 16
NW = NC * NS
EPW = E // NW
C1 = 50
NCH1 = EPW // C1
C2 = 100
NCH2 = EPW // C2
NP = 10112
RPT = NP // NS

RB = 1000
GRID = N // RB

W16 = 16


def _lrelu(v):
    return jnp.maximum(v, 0.2 * v)


def _tca_body(x_ref, w_ref, as_w_ref, ad_w_ref, h_ref, as_ref, ad_ref,
              maxs_ref, maxd_ref):
    i = pl.program_id(0)
    h = jnp.dot(x_ref[...], w_ref[...], preferred_element_type=jnp.float32)
    h_ref[...] = h
    a_s = jnp.dot(h, as_w_ref[...], preferred_element_type=jnp.float32)
    a_d = jnp.dot(h, ad_w_ref[...], preferred_element_type=jnp.float32)
    as_ref[...] = a_s
    ad_ref[...] = a_d
    bs = jnp.max(a_s, axis=0, keepdims=True)
    bd = jnp.max(a_d, axis=0, keepdims=True)

    @pl.when(i == 0)
    def _():
        maxs_ref[...] = bs
        maxd_ref[...] = bd

    @pl.when(i > 0)
    def _():
        maxs_ref[...] = jnp.maximum(maxs_ref[...], bs)
        maxd_ref[...] = jnp.maximum(maxd_ref[...], bd)


def _tca(x, w, as_w, ad_w):
    return pl.pallas_call(
        _tca_body,
        grid=(GRID,),
        in_specs=[
            pl.BlockSpec((RB, DIM), lambda i: (i, 0)),
            pl.BlockSpec((DIM, DIM), lambda i: (0, 0)),
            pl.BlockSpec((DIM, W16), lambda i: (0, 0)),
            pl.BlockSpec((DIM, W16), lambda i: (0, 0)),
        ],
        out_specs=[
            pl.BlockSpec((RB, DIM), lambda i: (i, 0)),
            pl.BlockSpec((RB, W16), lambda i: (i, 0)),
            pl.BlockSpec((RB, W16), lambda i: (i, 0)),
            pl.BlockSpec((1, W16), lambda i: (0, 0)),
            pl.BlockSpec((1, W16), lambda i: (0, 0)),
        ],
        out_shape=[
            jax.ShapeDtypeStruct((N, DIM), jnp.float32),
            jax.ShapeDtypeStruct((N, W16), jnp.float32),
            jax.ShapeDtypeStruct((N, W16), jnp.float32),
            jax.ShapeDtypeStruct((1, W16), jnp.float32),
            jax.ShapeDtypeStruct((1, W16), jnp.float32),
        ],
    )(x, w, as_w, ad_w)


def _sc0_body(src_hbm, z8_hbm, ones_hbm, deg_out,
              srcb, onesv, ssem, deg_sh):
    cid = lax.axis_index("c")
    sid = lax.axis_index("s")
    wid = sid * NC + cid

    r0 = sid * RPT
    pltpu.sync_copy(z8_hbm, deg_sh.at[pl.ds(r0, RPT)])
    pltpu.sync_copy(src_hbm.at[wid], srcb)
    pltpu.sync_copy(ones_hbm, onesv)
    plsc.subcore_barrier()

    def fire(i, _):
        pltpu.async_copy(onesv, deg_sh.at[srcb.at[i]], ssem, add=True)
        return ()

    def drain(i, _):
        pltpu.make_async_copy(onesv, deg_sh.at[srcb.at[i]], ssem).wait()
        return ()

    lax.fori_loop(0, NCH1, fire, (), unroll=False)
    lax.fori_loop(0, NCH1, drain, (), unroll=False)
    plsc.subcore_barrier()
    pltpu.sync_copy(deg_sh.at[pl.ds(r0, RPT)],
                    deg_out.at[cid, pl.ds(r0, RPT)])


def _sc0(src_r, z8, ones1):
    f = pl.kernel(
        _sc0_body,
        out_type=[jax.ShapeDtypeStruct((NC, NP, HEADS), jnp.float32)],
        mesh=plsc.VectorSubcoreMesh(core_axis_name="c", subcore_axis_name="s",
                                    num_cores=NC, num_subcores=NS),
        compiler_params=pltpu.CompilerParams(use_tc_tiling_on_sc=False),
        scratch_types=[
            pltpu.VMEM((NCH1, C1), jnp.int32),
            pltpu.VMEM((C1, HEADS), jnp.float32),
            pltpu.SemaphoreType.DMA,
            pltpu.VMEM_SHARED((NP, HEADS), jnp.float32),
        ],
    )
    return f(src_r, z8, ones1)[0]


def _sc1_body(src_hbm, dst_hbm, as_hbm, ad_hbm, h_hbm, bnd_hbm, z128_hbm,
              z16_hbm,
              numer_out, denom_out,
              srcb, dstb, asg0, asg1, adg0, adg1, exv0, exv1, hv0, hv1,
              bndv, gsem0, gsem1, ssem0, ssem1,
              numer_sh, denom_sh):
    cid = lax.axis_index("c")
    sid = lax.axis_index("s")
    wid = sid * NC + cid
    asg = (asg0, asg1)
    adg = (adg0, adg1)
    exv = (exv0, exv1)
    hv = (hv0, hv1)
    gsem = (gsem0, gsem1)
    ssem = (ssem0, ssem1)

    r0 = sid * RPT
    pltpu.sync_copy(z128_hbm, numer_sh.at[pl.ds(r0, RPT)])
    pltpu.sync_copy(z16_hbm, denom_sh.at[pl.ds(r0, RPT)])
    pltpu.sync_copy(src_hbm.at[wid], srcb)
    pltpu.sync_copy(dst_hbm.at[wid], dstb)
    pltpu.sync_copy(bnd_hbm, bndv)
    pltpu.sync_copy(z16_hbm.at[pl.ds(0, C1)], exv1)
    pltpu.sync_copy(z128_hbm.at[pl.ds(0, C1)], hv1)
    plsc.subcore_barrier()

    bnd = bndv[...]

    pltpu.async_copy(as_hbm.at[srcb.at[0]], asg0, gsem0)
    pltpu.async_copy(ad_hbm.at[dstb.at[0]], adg0, gsem0)
    pltpu.async_copy(h_hbm.at[srcb.at[0]], hv0, gsem0)
    pltpu.async_copy(exv1, denom_sh.at[dstb.at[0]], ssem1, add=True)
    pltpu.async_copy(hv1, numer_sh.at[dstb.at[0]], ssem1, add=True)

    def step(j, _):
        for b in range(2):
            i = 2 * j + b
            nb = 1 - b
            si = srcb.at[i]
            di = dstb.at[i]
            inx = jnp.minimum(i + 1, NCH1 - 1)
            sin = srcb.at[inx]
            din = dstb.at[inx]
            pltpu.make_async_copy(exv[nb], denom_sh.at[di], ssem[nb]).wait()
            pltpu.make_async_copy(hv[nb], numer_sh.at[di], ssem[nb]).wait()
            pltpu.async_copy(as_hbm.at[sin], asg[nb], gsem[nb])
            pltpu.async_copy(ad_hbm.at[din], adg[nb], gsem[nb])
            pltpu.async_copy(h_hbm.at[sin], hv[nb], gsem[nb])
            pltpu.make_async_copy(as_hbm.at[si], asg[b], gsem[b]).wait()
            pltpu.make_async_copy(ad_hbm.at[di], adg[b], gsem[b]).wait()
            pltpu.make_async_copy(h_hbm.at[si], hv[b], gsem[b]).wait()
            for r in range(C1):
                s = _lrelu(asg[b][r, :] + adg[b][r, :])
                exv[b][r, :] = jnp.exp(s - bnd)
            for e in range(C1):
                exrow = exv[b][e, :]
                for k in range(HEADS):
                    hv[b][e, pl.ds(k * DH, DH)] = (
                        hv[b][e, pl.ds(k * DH, DH)] * exrow[k])
            pltpu.async_copy(exv[b], denom_sh.at[di], ssem[b], add=True)
            pltpu.async_copy(hv[b], numer_sh.at[di], ssem[b], add=True)
        return ()

    lax.fori_loop(0, NCH1 // 2, step, (), unroll=False)
    pltpu.make_async_copy(as_hbm.at[srcb.at[0]], asg0, gsem0).wait()
    pltpu.make_async_copy(ad_hbm.at[dstb.at[0]], adg0, gsem0).wait()
    pltpu.make_async_copy(h_hbm.at[srcb.at[0]], hv0, gsem0).wait()
    pltpu.make_async_copy(exv1, denom_sh.at[dstb.at[0]], ssem1).wait()
    pltpu.make_async_copy(hv1, numer_sh.at[dstb.at[0]], ssem1).wait()
    plsc.subcore_barrier()
    pltpu.sync_copy(numer_sh.at[pl.ds(r0, RPT)],
                    numer_out.at[cid, pl.ds(r0, RPT)])
    pltpu.sync_copy(denom_sh.at[pl.ds(r0, RPT)],
                    denom_out.at[cid, pl.ds(r0, RPT)])


def _sc1(src_r, dst_r, as_tab, ad_tab, h, bnd16, z128, z16):
    f = pl.kernel(
        _sc1_body,
        out_type=[
            jax.ShapeDtypeStruct((NC, NP, DIM), jnp.float32),
            jax.ShapeDtypeStruct((NC, NP, W16), jnp.float32),
        ],
        mesh=plsc.VectorSubcoreMesh(core_axis_name="c", subcore_axis_name="s",
                                    num_cores=NC, num_subcores=NS),
        compiler_params=pltpu.CompilerParams(use_tc_tiling_on_sc=False),
        scratch_types=[
            pltpu.VMEM((NCH1, C1), jnp.int32),
            pltpu.VMEM((NCH1, C1), jnp.int32),
            pltpu.VMEM((C1, W16), jnp.float32),
            pltpu.VMEM((C1, W16), jnp.float32),
            pltpu.VMEM((C1, W16), jnp.float32),
            pltpu.VMEM((C1, W16), jnp.float32),
            pltpu.VMEM((C1, W16), jnp.float32),
            pltpu.VMEM((C1, W16), jnp.float32),
            pltpu.VMEM((C1, DIM), jnp.float32),
            pltpu.VMEM((C1, DIM), jnp.float32),
            pltpu.VMEM((W16,), jnp.float32),
            pltpu.SemaphoreType.DMA,
            pltpu.SemaphoreType.DMA,
            pltpu.SemaphoreType.DMA,
            pltpu.SemaphoreType.DMA,
            pltpu.VMEM_SHARED((NP, DIM), jnp.float32),
            pltpu.VMEM_SHARED((NP, W16), jnp.float32),
        ],
    )
    return f(src_r, dst_r, as_tab, ad_tab, h, bnd16, z128, z16)


def _sc2_body(src_hbm, dst_hbm, tab_hbm, z16_hbm, acc_out,
              srcb, dstb, tv0, tv1, tv2, tv3,
              gsem0, gsem1, gsem2, gsem3, ssem0, ssem1, ssem2, ssem3,
              acc_sh):
    cid = lax.axis_index("c")
    sid = lax.axis_index("s")
    wid = sid * NC + cid
    tv = (tv0, tv1, tv2, tv3)
    gsem = (gsem0, gsem1, gsem2, gsem3)
    ssem = (ssem0, ssem1, ssem2, ssem3)

    r0 = sid * RPT
    pltpu.sync_copy(z16_hbm, acc_sh.at[pl.ds(r0, RPT)])
    pltpu.sync_copy(src_hbm.at[wid], srcb)
    pltpu.sync_copy(dst_hbm.at[wid], dstb)
    pltpu.sync_copy(z16_hbm.at[pl.ds(0, C2)], tv2)
    pltpu.sync_copy(z16_hbm.at[pl.ds(0, C2)], tv3)
    plsc.subcore_barrier()

    pltpu.async_copy(tab_hbm.at[dstb.at[0]], tv0, gsem0)
    pltpu.async_copy(tab_hbm.at[dstb.at[1]], tv1, gsem1)
    pltpu.async_copy(tv2, acc_sh.at[srcb.at[0]], ssem2, add=True)
    pltpu.async_copy(tv3, acc_sh.at[srcb.at[0]], ssem3, add=True)

    def step(j, _):
        for b in range(4):
            i = 4 * j + b
            pb = (b + 2) % 4
            si = srcb.at[i]
            inx = jnp.minimum(i + 2, NCH2 - 1)
            pltpu.make_async_copy(tv[pb], acc_sh.at[si], ssem[pb]).wait()
            pltpu.async_copy(tab_hbm.at[dstb.at[inx]], tv[pb], gsem[pb])
            pltpu.make_async_copy(tab_hbm.at[dstb.at[i]], tv[b], gsem[b]).wait()
            pltpu.async_copy(tv[b], acc_sh.at[si], ssem[b], add=True)
        return ()

    lax.fori_loop(0, NCH2 // 4, step, (), unroll=False)
    pltpu.make_async_copy(tab_hbm.at[dstb.at[0]], tv0, gsem0).wait()
    pltpu.make_async_copy(tab_hbm.at[dstb.at[0]], tv1, gsem1).wait()
    pltpu.make_async_copy(tv2, acc_sh.at[srcb.at[0]], ssem2).wait()
    pltpu.make_async_copy(tv3, acc_sh.at[srcb.at[0]], ssem3).wait()
    plsc.subcore_barrier()
    pltpu.sync_copy(acc_sh.at[pl.ds(r0, RPT)],
                    acc_out.at[cid, pl.ds(r0, RPT)])


def _sc2(src_r, dst_r, tab, z16):
    f = pl.kernel(
        _sc2_body,
        out_type=[jax.ShapeDtypeStruct((NC, NP, W16), jnp.float32)],
        mesh=plsc.VectorSubcoreMesh(core_axis_name="c", subcore_axis_name="s",
                                    num_cores=NC, num_subcores=NS),
        compiler_params=pltpu.CompilerParams(use_tc_tiling_on_sc=False),
        scratch_types=[
            pltpu.VMEM((NCH2, C2), jnp.int32),
            pltpu.VMEM((NCH2, C2), jnp.int32),
            pltpu.VMEM((C2, W16), jnp.float32),
            pltpu.VMEM((C2, W16), jnp.float32),
            pltpu.VMEM((C2, W16), jnp.float32),
            pltpu.VMEM((C2, W16), jnp.float32),
            pltpu.SemaphoreType.DMA,
            pltpu.SemaphoreType.DMA,
            pltpu.SemaphoreType.DMA,
            pltpu.SemaphoreType.DMA,
            pltpu.SemaphoreType.DMA,
            pltpu.SemaphoreType.DMA,
            pltpu.SemaphoreType.DMA,
            pltpu.SemaphoreType.DMA,
            pltpu.VMEM_SHARED((NP, W16), jnp.float32),
        ],
    )
    return f(src_r, dst_r, tab, z16)[0]


def _tcb_body(nump_ref, denp_ref, degp_ref, h_ref, as_ref, ad_ref, bnd_ref,
              r_ref, bias_ref, xh_ref, t1_ref, dis_ref, m_ref, msq_ref):
    ss = as_ref[:, :HEADS] + ad_ref[:, :HEADS]
    ex_self = jnp.exp(_lrelu(ss) - bnd_ref[:, :HEADS])
    den8 = denp_ref[0][:, :HEADS] + denp_ref[1][:, :HEADS] + ex_self + 1e-16
    denx = jnp.dot(den8, r_ref[...], preferred_element_type=jnp.float32)
    numx = (nump_ref[0] + nump_ref[1]
            + h_ref[...] * jnp.dot(ex_self, r_ref[...],
                                   preferred_element_type=jnp.float32))
    xh = numx / denx + bias_ref[...]
    xh_ref[...] = xh
    m = jnp.mean(xh, axis=1, keepdims=True)
    msq = jnp.mean(xh * xh, axis=1, keepdims=True)
    deg = degp_ref[0][:, 0:1] + degp_ref[1][:, 0:1]
    dis = lax.rsqrt(deg + EPS)
    dis_ref[...] = dis
    m_ref[...] = m
    msq_ref[...] = msq
    col = lax.broadcasted_iota(jnp.int32, (RB, W16), 1)
    t1a = jnp.broadcast_to(dis * m, (RB, W16))
    t1b = jnp.broadcast_to(dis * msq, (RB, W16))
    t1_ref[...] = jnp.where(col == 0, t1a, jnp.where(col == 1, t1b, 0.0))


def _tcb(nump, denp, degp, h, as_tab, ad_tab, bnd8, r_exp, bias_row):
    return pl.pallas_call(
        _tcb_body,
        grid=(GRID,),
        in_specs=[
            pl.BlockSpec((NC, RB, DIM), lambda i: (0, i, 0)),
            pl.BlockSpec((NC, RB, W16), lambda i: (0, i, 0)),
            pl.BlockSpec((NC, RB, HEADS), lambda i: (0, i, 0)),
            pl.BlockSpec((RB, DIM), lambda i: (i, 0)),
            pl.BlockSpec((RB, W16), lambda i: (i, 0)),
            pl.BlockSpec((RB, W16), lambda i: (i, 0)),
            pl.BlockSpec((1, W16), lambda i: (0, 0)),
            pl.BlockSpec((HEADS, DIM), lambda i: (0, 0)),
            pl.BlockSpec((1, DIM), lambda i: (0, 0)),
        ],
        out_specs=[
            pl.BlockSpec((RB, DIM), lambda i: (i, 0)),
            pl.BlockSpec((RB, W16), lambda i: (i, 0)),
            pl.BlockSpec((RB, 1), lambda i: (i, 0)),
            pl.BlockSpec((RB, 1), lambda i: (i, 0)),
            pl.BlockSpec((RB, 1), lambda i: (i, 0)),
        ],
        out_shape=[
            jax.ShapeDtypeStruct((N, DIM), jnp.float32),
            jax.ShapeDtypeStruct((N, W16), jnp.float32),
            jax.ShapeDtypeStruct((N, 1), jnp.float32),
            jax.ShapeDtypeStruct((N, 1), jnp.float32),
            jax.ShapeDtypeStruct((N, 1), jnp.float32),
        ],
    )(nump, denp, degp, h, as_tab, ad_tab, bnd8, r_exp, bias_row)


def _tcb2_body(acc1_ref, dis_ref, m_ref, t2_ref, mean_ref, bv_ref):
    a1 = acc1_ref[0] + acc1_ref[1]
    dis = dis_ref[...]
    mean = dis * a1[:, 0:1]
    bv = dis * a1[:, 1:2]
    mean_ref[...] = mean
    bv_ref[...] = bv
    t2a = dis * (2.0 * mean * m_ref[...] - mean * mean)
    col = lax.broadcasted_iota(jnp.int32, (RB, W16), 1)
    t2_ref[...] = jnp.where(col == 0, jnp.broadcast_to(t2a, (RB, W16)), 0.0)


def _tcb2(acc1, dis, m):
    return pl.pallas_call(
        _tcb2_body,
        grid=(GRID,),
        in_specs=[
            pl.BlockSpec((NC, RB, W16), lambda i: (0, i, 0)),
            pl.BlockSpec((RB, 1), lambda i: (i, 0)),
            pl.BlockSpec((RB, 1), lambda i: (i, 0)),
        ],
        out_specs=[
            pl.BlockSpec((RB, W16), lambda i: (i, 0)),
            pl.BlockSpec((RB, 1), lambda i: (i, 0)),
            pl.BlockSpec((RB, 1), lambda i: (i, 0)),
        ],
        out_shape=[
            jax.ShapeDtypeStruct((N, W16), jnp.float32),
            jax.ShapeDtypeStruct((N, 1), jnp.float32),
            jax.ShapeDtypeStruct((N, 1), jnp.float32),
        ],
    )(acc1, dis, m)


def _tcc_body(xh_ref, x_ref, mean_ref, bv_ref, dis_ref, acc2_ref,
              lsc_ref, lbi_ref, lw_ref, lb_ref, w1_ref, b1_ref, w2_ref,
              b2_ref, out_ref):
    a2 = acc2_ref[0][:, 0:1] + acc2_ref[1][:, 0:1]
    var = bv_ref[...] - dis_ref[...] * a2
    xh = xh_ref[...]
    xn = (xh - mean_ref[...]) / jnp.sqrt(var + EPS) * lsc_ref[...] + lbi_ref[...]
    out = xn + x_ref[...]
    mu = jnp.mean(out, axis=1, keepdims=True)
    d = out - mu
    vv = jnp.mean(d * d, axis=1, keepdims=True)
    o1 = d / jnp.sqrt(vv + 1e-5) * lw_ref[...] + lb_ref[...]
    hpre = jnp.dot(o1, w1_ref[...], preferred_element_type=jnp.float32) + b1_ref[...]
    hh = 0.5 * hpre * (1.0 + lax.erf(hpre * 0.7071067811865476))
    o2 = jnp.dot(hh, w2_ref[...], preferred_element_type=jnp.float32) + b2_ref[...] + o1
    nrm = jnp.sqrt(jnp.sum(o2 * o2, axis=1, keepdims=True))
    out_ref[...] = o2 / jnp.maximum(nrm, 1e-12)


def _tcc(xh, x, mean, bv, dis, acc2, lsc, lbi, lw, lb, w1, b1, w2, b2):
    return pl.pallas_call(
        _tcc_body,
        grid=(GRID,),
        in_specs=[
            pl.BlockSpec((RB, DIM), lambda i: (i, 0)),
            pl.BlockSpec((RB, DIM), lambda i: (i, 0)),
            pl.BlockSpec((RB, 1), lambda i: (i, 0)),
            pl.BlockSpec((RB, 1), lambda i: (i, 0)),
            pl.BlockSpec((RB, 1), lambda i: (i, 0)),
            pl.BlockSpec((NC, RB, W16), lambda i: (0, i, 0)),
            pl.BlockSpec((1, DIM), lambda i: (0, 0)),
            pl.BlockSpec((1, DIM), lambda i: (0, 0)),
            pl.BlockSpec((1, DIM), lambda i: (0, 0)),
            pl.BlockSpec((1, DIM), lambda i: (0, 0)),
            pl.BlockSpec((DIM, 2 * DIM), lambda i: (0, 0)),
            pl.BlockSpec((1, 2 * DIM), lambda i: (0, 0)),
            pl.BlockSpec((2 * DIM, DIM), lambda i: (0, 0)),
            pl.BlockSpec((1, DIM), lambda i: (0, 0)),
        ],
        out_specs=pl.BlockSpec((RB, DIM), lambda i: (i, 0)),
        out_shape=jax.ShapeDtypeStruct((N, DIM), jnp.float32),
    )(xh, x, mean, bv, dis, acc2, lsc, lbi, lw, lb, w1, b1, w2, b2)


def kernel(x, edge_index, W_gat, att_src, att_dst, bias_gat, lap_scale,
           lap_bias, ln2_w, ln2_b, W1, b1, W2, b2):
    src = edge_index[0].astype(jnp.int32)
    dst = edge_index[1].astype(jnp.int32)
    src1 = src.reshape(NW, NCH1, C1)
    dst1 = dst.reshape(NW, NCH1, C1)
    src2 = src.reshape(NW, NCH2, C2)
    dst2 = dst.reshape(NW, NCH2, C2)

    lanes = jnp.arange(DIM, dtype=jnp.int32)
    hsel = (lanes[:, None] // DH) == (jnp.arange(W16, dtype=jnp.int32) % HEADS)[None, :]
    as_w = jnp.where(hsel, att_src.reshape(DIM)[:, None], 0.0)
    ad_w = jnp.where(hsel, att_dst.reshape(DIM)[:, None], 0.0)
    sel8 = (lanes[:, None] // DH) == jnp.arange(HEADS, dtype=jnp.int32)[None, :]
    r_exp = sel8.T.astype(jnp.float32)

    h, as_tab, ad_tab, maxs, maxd = _tca(x, W_gat, as_w, ad_w)
    bnd = _lrelu(maxs + maxd)
    bnd16 = bnd.reshape(W16)

    z128 = jnp.zeros((RPT, DIM), jnp.float32)
    z16 = jnp.zeros((RPT, W16), jnp.float32)
    z8 = jnp.zeros((RPT, HEADS), jnp.float32)
    ones1 = jnp.ones((C1, HEADS), jnp.float32)

    degp = _sc0(src1, z8, ones1)
    nump, denp = _sc1(src1, dst1, as_tab, ad_tab, h, bnd16, z128, z16)

    xh, t1, dis, m, msq = _tcb(nump, denp, degp, h, as_tab, ad_tab, bnd,
                               r_exp, bias_gat.reshape(1, DIM))

    acc1 = _sc2(src2, dst2, t1, z16)
    t2, mean, bv = _tcb2(acc1, dis, m)
    acc2 = _sc2(src2, dst2, t2, z16)

    return _tcc(xh, x, mean, bv, dis, acc2,
                lap_scale.reshape(1, DIM), lap_bias.reshape(1, DIM),
                ln2_w.reshape(1, DIM), ln2_b.reshape(1, DIM),
                W1, b1.reshape(1, 2 * DIM), W2, b2.reshape(1, DIM))


# speedup vs baseline: 1.0863x; 1.0154x over previous
import jax
import jax.numpy as jnp
from jax import lax
from jax.experimental import pallas as pl
from jax.experimental.pallas import tpu as pltpu
from jax.experimental.pallas import tpu_sc as plsc

N = 10000
E = 320000
DIM = 128
HEADS = 8
DH = 16
EPS = 1e-6

NC = 2
NS = 16
NW = NC * NS
EPW = E // NW
C1 = 50
NCH1 = EPW // C1
C2 = 125
NCH2 = EPW // C2
NP = 10112
RPT = NP // NS

RB = 1000
GRID = N // RB

W16 = 16


def _lrelu(v):
    return jnp.maximum(v, 0.2 * v)


def _tca_body(x_ref, w_ref, as_w_ref, ad_w_ref, h_ref, as_ref, ad_ref,
              maxs_ref, maxd_ref):
    i = pl.program_id(0)
    h = jnp.dot(x_ref[...], w_ref[...], preferred_element_type=jnp.float32)
    h_ref[...] = h
    a_s = jnp.dot(h, as_w_ref[...], preferred_element_type=jnp.float32)
    a_d = jnp.dot(h, ad_w_ref[...], preferred_element_type=jnp.float32)
    as_ref[...] = a_s
    ad_ref[...] = a_d
    bs = jnp.max(a_s, axis=0, keepdims=True)
    bd = jnp.max(a_d, axis=0, keepdims=True)

    @pl.when(i == 0)
    def _():
        maxs_ref[...] = bs
        maxd_ref[...] = bd

    @pl.when(i > 0)
    def _():
        maxs_ref[...] = jnp.maximum(maxs_ref[...], bs)
        maxd_ref[...] = jnp.maximum(maxd_ref[...], bd)


def _tca(x, w, as_w, ad_w):
    return pl.pallas_call(
        _tca_body,
        grid=(GRID,),
        in_specs=[
            pl.BlockSpec((RB, DIM), lambda i: (i, 0)),
            pl.BlockSpec((DIM, DIM), lambda i: (0, 0)),
            pl.BlockSpec((DIM, W16), lambda i: (0, 0)),
            pl.BlockSpec((DIM, W16), lambda i: (0, 0)),
        ],
        out_specs=[
            pl.BlockSpec((RB, DIM), lambda i: (i, 0)),
            pl.BlockSpec((RB, W16), lambda i: (i, 0)),
            pl.BlockSpec((RB, W16), lambda i: (i, 0)),
            pl.BlockSpec((1, W16), lambda i: (0, 0)),
            pl.BlockSpec((1, W16), lambda i: (0, 0)),
        ],
        out_shape=[
            jax.ShapeDtypeStruct((N, DIM), jnp.float32),
            jax.ShapeDtypeStruct((N, W16), jnp.float32),
            jax.ShapeDtypeStruct((N, W16), jnp.float32),
            jax.ShapeDtypeStruct((1, W16), jnp.float32),
            jax.ShapeDtypeStruct((1, W16), jnp.float32),
        ],
    )(x, w, as_w, ad_w)


def _sc0_body(src_hbm, z8_hbm, ones_hbm, deg_out,
              srcb, onesv, ssem, deg_sh):
    cid = lax.axis_index("c")
    sid = lax.axis_index("s")
    wid = sid * NC + cid

    r0 = sid * RPT
    pltpu.sync_copy(z8_hbm, deg_sh.at[pl.ds(r0, RPT)])
    pltpu.sync_copy(src_hbm.at[wid], srcb)
    pltpu.sync_copy(ones_hbm, onesv)
    plsc.subcore_barrier()

    def fire(i, _):
        pltpu.async_copy(onesv, deg_sh.at[srcb.at[i]], ssem, add=True)
        return ()

    def drain(i, _):
        pltpu.make_async_copy(onesv, deg_sh.at[srcb.at[i]], ssem).wait()
        return ()

    lax.fori_loop(0, NCH1, fire, (), unroll=False)
    lax.fori_loop(0, NCH1, drain, (), unroll=False)
    plsc.subcore_barrier()
    pltpu.sync_copy(deg_sh.at[pl.ds(r0, RPT)],
                    deg_out.at[cid, pl.ds(r0, RPT)])


def _sc0(src_r, z8, ones1):
    f = pl.kernel(
        _sc0_body,
        out_type=[jax.ShapeDtypeStruct((NC, NP, HEADS), jnp.float32)],
        mesh=plsc.VectorSubcoreMesh(core_axis_name="c", subcore_axis_name="s",
                                    num_cores=NC, num_subcores=NS),
        compiler_params=pltpu.CompilerParams(use_tc_tiling_on_sc=False),
        scratch_types=[
            pltpu.VMEM((NCH1, C1), jnp.int32),
            pltpu.VMEM((C1, HEADS), jnp.float32),
            pltpu.SemaphoreType.DMA,
            pltpu.VMEM_SHARED((NP, HEADS), jnp.float32),
        ],
    )
    return f(src_r, z8, ones1)[0]


def _sc1_body(src_hbm, dst_hbm, as_hbm, ad_hbm, h_hbm, bnd_hbm, z128_hbm,
              z16_hbm,
              numer_out, denom_out,
              srcb, dstb, asg0, asg1, adg0, adg1, exv0, exv1, hv0, hv1,
              bndv, gsem0, gsem1, ssem0, ssem1,
              numer_sh, denom_sh):
    cid = lax.axis_index("c")
    sid = lax.axis_index("s")
    wid = sid * NC + cid
    asg = (asg0, asg1)
    adg = (adg0, adg1)
    exv = (exv0, exv1)
    hv = (hv0, hv1)
    gsem = (gsem0, gsem1)
    ssem = (ssem0, ssem1)

    r0 = sid * RPT
    pltpu.sync_copy(z128_hbm, numer_sh.at[pl.ds(r0, RPT)])
    pltpu.sync_copy(z16_hbm, denom_sh.at[pl.ds(r0, RPT)])
    pltpu.sync_copy(src_hbm.at[wid], srcb)
    pltpu.sync_copy(dst_hbm.at[wid], dstb)
    pltpu.sync_copy(bnd_hbm, bndv)
    pltpu.sync_copy(z16_hbm.at[pl.ds(0, C1)], exv1)
    pltpu.sync_copy(z128_hbm.at[pl.ds(0, C1)], hv1)
    plsc.subcore_barrier()

    bnd = bndv[...]

    pltpu.async_copy(as_hbm.at[srcb.at[0]], asg0, gsem0)
    pltpu.async_copy(ad_hbm.at[dstb.at[0]], adg0, gsem0)
    pltpu.async_copy(h_hbm.at[srcb.at[0]], hv0, gsem0)
    pltpu.async_copy(exv1, denom_sh.at[dstb.at[0]], ssem1, add=True)
    pltpu.async_copy(hv1, numer_sh.at[dstb.at[0]], ssem1, add=True)

    def step(j, _):
        for b in range(2):
            i = 2 * j + b
            nb = 1 - b
            si = srcb.at[i]
            di = dstb.at[i]
            inx = jnp.minimum(i + 1, NCH1 - 1)
            sin = srcb.at[inx]
            din = dstb.at[inx]
            pltpu.make_async_copy(exv[nb], denom_sh.at[di], ssem[nb]).wait()
            pltpu.make_async_copy(hv[nb], numer_sh.at[di], ssem[nb]).wait()
            pltpu.async_copy(as_hbm.at[sin], asg[nb], gsem[nb])
            pltpu.async_copy(ad_hbm.at[din], adg[nb], gsem[nb])
            pltpu.async_copy(h_hbm.at[sin], hv[nb], gsem[nb])
            pltpu.make_async_copy(as_hbm.at[si], asg[b], gsem[b]).wait()
            pltpu.make_async_copy(ad_hbm.at[di], adg[b], gsem[b]).wait()
            pltpu.make_async_copy(h_hbm.at[si], hv[b], gsem[b]).wait()
            for r in range(C1):
                s = _lrelu(asg[b][r, :] + adg[b][r, :])
                exv[b][r, :] = jnp.exp(s - bnd)
            for e in range(C1):
                exrow = exv[b][e, :]
                for k in range(HEADS):
                    hv[b][e, pl.ds(k * DH, DH)] = (
                        hv[b][e, pl.ds(k * DH, DH)] * exrow[k])
            pltpu.async_copy(exv[b], denom_sh.at[di], ssem[b], add=True)
            pltpu.async_copy(hv[b], numer_sh.at[di], ssem[b], add=True)
        return ()

    lax.fori_loop(0, NCH1 // 2, step, (), unroll=False)
    pltpu.make_async_copy(as_hbm.at[srcb.at[0]], asg0, gsem0).wait()
    pltpu.make_async_copy(ad_hbm.at[dstb.at[0]], adg0, gsem0).wait()
    pltpu.make_async_copy(h_hbm.at[srcb.at[0]], hv0, gsem0).wait()
    pltpu.make_async_copy(exv1, denom_sh.at[dstb.at[0]], ssem1).wait()
    pltpu.make_async_copy(hv1, numer_sh.at[dstb.at[0]], ssem1).wait()
    plsc.subcore_barrier()
    pltpu.sync_copy(numer_sh.at[pl.ds(r0, RPT)],
                    numer_out.at[cid, pl.ds(r0, RPT)])
    pltpu.sync_copy(denom_sh.at[pl.ds(r0, RPT)],
                    denom_out.at[cid, pl.ds(r0, RPT)])


def _sc1(src_r, dst_r, as_tab, ad_tab, h, bnd16, z128, z16):
    f = pl.kernel(
        _sc1_body,
        out_type=[
            jax.ShapeDtypeStruct((NC, NP, DIM), jnp.float32),
            jax.ShapeDtypeStruct((NC, NP, W16), jnp.float32),
        ],
        mesh=plsc.VectorSubcoreMesh(core_axis_name="c", subcore_axis_name="s",
                                    num_cores=NC, num_subcores=NS),
        compiler_params=pltpu.CompilerParams(use_tc_tiling_on_sc=False),
        scratch_types=[
            pltpu.VMEM((NCH1, C1), jnp.int32),
            pltpu.VMEM((NCH1, C1), jnp.int32),
            pltpu.VMEM((C1, W16), jnp.float32),
            pltpu.VMEM((C1, W16), jnp.float32),
            pltpu.VMEM((C1, W16), jnp.float32),
            pltpu.VMEM((C1, W16), jnp.float32),
            pltpu.VMEM((C1, W16), jnp.float32),
            pltpu.VMEM((C1, W16), jnp.float32),
            pltpu.VMEM((C1, DIM), jnp.float32),
            pltpu.VMEM((C1, DIM), jnp.float32),
            pltpu.VMEM((W16,), jnp.float32),
            pltpu.SemaphoreType.DMA,
            pltpu.SemaphoreType.DMA,
            pltpu.SemaphoreType.DMA,
            pltpu.SemaphoreType.DMA,
            pltpu.VMEM_SHARED((NP, DIM), jnp.float32),
            pltpu.VMEM_SHARED((NP, W16), jnp.float32),
        ],
    )
    return f(src_r, dst_r, as_tab, ad_tab, h, bnd16, z128, z16)


def _sc2_body(src_hbm, dst_hbm, tab_hbm, z16_hbm, acc_out,
              srcb, dstb, tv0, tv1, tv2, tv3,
              gsem0, gsem1, gsem2, gsem3, ssem0, ssem1, ssem2, ssem3,
              acc_sh):
    cid = lax.axis_index("c")
    sid = lax.axis_index("s")
    wid = sid * NC + cid
    tv = (tv0, tv1, tv2, tv3)
    gsem = (gsem0, gsem1, gsem2, gsem3)
    ssem = (ssem0, ssem1, ssem2, ssem3)

    r0 = sid * RPT
    pltpu.sync_copy(z16_hbm, acc_sh.at[pl.ds(r0, RPT)])
    pltpu.sync_copy(src_hbm.at[wid], srcb)
    pltpu.sync_copy(dst_hbm.at[wid], dstb)
    pltpu.sync_copy(z16_hbm.at[pl.ds(0, C2)], tv2)
    pltpu.sync_copy(z16_hbm.at[pl.ds(0, C2)], tv3)
    plsc.subcore_barrier()

    pltpu.async_copy(tab_hbm.at[dstb.at[0]], tv0, gsem0)
    pltpu.async_copy(tab_hbm.at[dstb.at[1]], tv1, gsem1)
    pltpu.async_copy(tv2, acc_sh.at[srcb.at[0]], ssem2, add=True)
    pltpu.async_copy(tv3, acc_sh.at[srcb.at[0]], ssem3, add=True)

    def step(j, _):
        for b in range(4):
            i = 4 * j + b
            pb = (b + 2) % 4
            si = srcb.at[i]
            inx = jnp.minimum(i + 2, NCH2 - 1)
            pltpu.make_async_copy(tv[pb], acc_sh.at[si], ssem[pb]).wait()
            pltpu.async_copy(tab_hbm.at[dstb.at[inx]], tv[pb], gsem[pb])
            pltpu.make_async_copy(tab_hbm.at[dstb.at[i]], tv[b], gsem[b]).wait()
            pltpu.async_copy(tv[b], acc_sh.at[si], ssem[b], add=True)
        return ()

    lax.fori_loop(0, NCH2 // 4, step, (), unroll=False)
    pltpu.make_async_copy(tab_hbm.at[dstb.at[0]], tv0, gsem0).wait()
    pltpu.make_async_copy(tab_hbm.at[dstb.at[0]], tv1, gsem1).wait()
    pltpu.make_async_copy(tv2, acc_sh.at[srcb.at[0]], ssem2).wait()
    pltpu.make_async_copy(tv3, acc_sh.at[srcb.at[0]], ssem3).wait()
    plsc.subcore_barrier()
    pltpu.sync_copy(acc_sh.at[pl.ds(r0, RPT)],
                    acc_out.at[cid, pl.ds(r0, RPT)])


def _sc2(src_r, dst_r, tab, z16):
    f = pl.kernel(
        _sc2_body,
        out_type=[jax.ShapeDtypeStruct((NC, NP, W16), jnp.float32)],
        mesh=plsc.VectorSubcoreMesh(core_axis_name="c", subcore_axis_name="s",
                                    num_cores=NC, num_subcores=NS),
        compiler_params=pltpu.CompilerParams(use_tc_tiling_on_sc=False),
        scratch_types=[
            pltpu.VMEM((NCH2, C2), jnp.int32),
            pltpu.VMEM((NCH2, C2), jnp.int32),
            pltpu.VMEM((C2, W16), jnp.float32),
            pltpu.VMEM((C2, W16), jnp.float32),
            pltpu.VMEM((C2, W16), jnp.float32),
            pltpu.VMEM((C2, W16), jnp.float32),
            pltpu.SemaphoreType.DMA,
            pltpu.SemaphoreType.DMA,
            pltpu.SemaphoreType.DMA,
            pltpu.SemaphoreType.DMA,
            pltpu.SemaphoreType.DMA,
            pltpu.SemaphoreType.DMA,
            pltpu.SemaphoreType.DMA,
            pltpu.SemaphoreType.DMA,
            pltpu.VMEM_SHARED((NP, W16), jnp.float32),
        ],
    )
    return f(src_r, dst_r, tab, z16)[0]


def _tcb_body(nump_ref, denp_ref, degp_ref, h_ref, as_ref, ad_ref, bnd_ref,
              r_ref, bias_ref, xh_ref, t1_ref, dis_ref, m_ref, msq_ref):
    ss = as_ref[:, :HEADS] + ad_ref[:, :HEADS]
    ex_self = jnp.exp(_lrelu(ss) - bnd_ref[:, :HEADS])
    den8 = denp_ref[0][:, :HEADS] + denp_ref[1][:, :HEADS] + ex_self + 1e-16
    denx = jnp.dot(den8, r_ref[...], preferred_element_type=jnp.float32)
    numx = (nump_ref[0] + nump_ref[1]
            + h_ref[...] * jnp.dot(ex_self, r_ref[...],
                                   preferred_element_type=jnp.float32))
    xh = numx / denx + bias_ref[...]
    xh_ref[...] = xh
    m = jnp.mean(xh, axis=1, keepdims=True)
    msq = jnp.mean(xh * xh, axis=1, keepdims=True)
    deg = degp_ref[0][:, 0:1] + degp_ref[1][:, 0:1]
    dis = lax.rsqrt(deg + EPS)
    dis_ref[...] = dis
    m_ref[...] = m
    msq_ref[...] = msq
    col = lax.broadcasted_iota(jnp.int32, (RB, W16), 1)
    t1a = jnp.broadcast_to(dis * m, (RB, W16))
    t1b = jnp.broadcast_to(dis * msq, (RB, W16))
    t1_ref[...] = jnp.where(col == 0, t1a, jnp.where(col == 1, t1b, 0.0))


def _tcb(nump, denp, degp, h, as_tab, ad_tab, bnd8, r_exp, bias_row):
    return pl.pallas_call(
        _tcb_body,
        grid=(GRID,),
        in_specs=[
            pl.BlockSpec((NC, RB, DIM), lambda i: (0, i, 0)),
            pl.BlockSpec((NC, RB, W16), lambda i: (0, i, 0)),
            pl.BlockSpec((NC, RB, HEADS), lambda i: (0, i, 0)),
            pl.BlockSpec((RB, DIM), lambda i: (i, 0)),
            pl.BlockSpec((RB, W16), lambda i: (i, 0)),
            pl.BlockSpec((RB, W16), lambda i: (i, 0)),
            pl.BlockSpec((1, W16), lambda i: (0, 0)),
            pl.BlockSpec((HEADS, DIM), lambda i: (0, 0)),
            pl.BlockSpec((1, DIM), lambda i: (0, 0)),
        ],
        out_specs=[
            pl.BlockSpec((RB, DIM), lambda i: (i, 0)),
            pl.BlockSpec((RB, W16), lambda i: (i, 0)),
            pl.BlockSpec((RB, 1), lambda i: (i, 0)),
            pl.BlockSpec((RB, 1), lambda i: (i, 0)),
            pl.BlockSpec((RB, 1), lambda i: (i, 0)),
        ],
        out_shape=[
            jax.ShapeDtypeStruct((N, DIM), jnp.float32),
            jax.ShapeDtypeStruct((N, W16), jnp.float32),
            jax.ShapeDtypeStruct((N, 1), jnp.float32),
            jax.ShapeDtypeStruct((N, 1), jnp.float32),
            jax.ShapeDtypeStruct((N, 1), jnp.float32),
        ],
    )(nump, denp, degp, h, as_tab, ad_tab, bnd8, r_exp, bias_row)


def _tcb2_body(acc1_ref, dis_ref, m_ref, t2_ref, mean_ref, bv_ref):
    a1 = acc1_ref[0] + acc1_ref[1]
    dis = dis_ref[...]
    mean = dis * a1[:, 0:1]
    bv = dis * a1[:, 1:2]
    mean_ref[...] = mean
    bv_ref[...] = bv
    t2a = dis * (2.0 * mean * m_ref[...] - mean * mean)
    col = lax.broadcasted_iota(jnp.int32, (RB, W16), 1)
    t2_ref[...] = jnp.where(col == 0, jnp.broadcast_to(t2a, (RB, W16)), 0.0)


def _tcb2(acc1, dis, m):
    return pl.pallas_call(
        _tcb2_body,
        grid=(GRID,),
        in_specs=[
            pl.BlockSpec((NC, RB, W16), lambda i: (0, i, 0)),
            pl.BlockSpec((RB, 1), lambda i: (i, 0)),
            pl.BlockSpec((RB, 1), lambda i: (i, 0)),
        ],
        out_specs=[
            pl.BlockSpec((RB, W16), lambda i: (i, 0)),
            pl.BlockSpec((RB, 1), lambda i: (i, 0)),
            pl.BlockSpec((RB, 1), lambda i: (i, 0)),
        ],
        out_shape=[
            jax.ShapeDtypeStruct((N, W16), jnp.float32),
            jax.ShapeDtypeStruct((N, 1), jnp.float32),
            jax.ShapeDtypeStruct((N, 1), jnp.float32),
        ],
    )(acc1, dis, m)


def _tcc_body(xh_ref, x_ref, mean_ref, bv_ref, dis_ref, acc2_ref,
              lsc_ref, lbi_ref, lw_ref, lb_ref, w1_ref, b1_ref, w2_ref,
              b2_ref, out_ref):
    a2 = acc2_ref[0][:, 0:1] + acc2_ref[1][:, 0:1]
    var = bv_ref[...] - dis_ref[...] * a2
    xh = xh_ref[...]
    xn = (xh - mean_ref[...]) / jnp.sqrt(var + EPS) * lsc_ref[...] + lbi_ref[...]
    out = xn + x_ref[...]
    mu = jnp.mean(out, axis=1, keepdims=True)
    d = out - mu
    vv = jnp.mean(d * d, axis=1, keepdims=True)
    o1 = d / jnp.sqrt(vv + 1e-5) * lw_ref[...] + lb_ref[...]
    hpre = jnp.dot(o1, w1_ref[...], preferred_element_type=jnp.float32) + b1_ref[...]
    hh = 0.5 * hpre * (1.0 + lax.erf(hpre * 0.7071067811865476))
    o2 = jnp.dot(hh, w2_ref[...], preferred_element_type=jnp.float32) + b2_ref[...] + o1
    nrm = jnp.sqrt(jnp.sum(o2 * o2, axis=1, keepdims=True))
    out_ref[...] = o2 / jnp.maximum(nrm, 1e-12)


def _tcc(xh, x, mean, bv, dis, acc2, lsc, lbi, lw, lb, w1, b1, w2, b2):
    return pl.pallas_call(
        _tcc_body,
        grid=(GRID,),
        in_specs=[
            pl.BlockSpec((RB, DIM), lambda i: (i, 0)),
            pl.BlockSpec((RB, DIM), lambda i: (i, 0)),
            pl.BlockSpec((RB, 1), lambda i: (i, 0)),
            pl.BlockSpec((RB, 1), lambda i: (i, 0)),
            pl.BlockSpec((RB, 1), lambda i: (i, 0)),
            pl.BlockSpec((NC, RB, W16), lambda i: (0, i, 0)),
            pl.BlockSpec((1, DIM), lambda i: (0, 0)),
            pl.BlockSpec((1, DIM), lambda i: (0, 0)),
            pl.BlockSpec((1, DIM), lambda i: (0, 0)),
            pl.BlockSpec((1, DIM), lambda i: (0, 0)),
            pl.BlockSpec((DIM, 2 * DIM), lambda i: (0, 0)),
            pl.BlockSpec((1, 2 * DIM), lambda i: (0, 0)),
            pl.BlockSpec((2 * DIM, DIM), lambda i: (0, 0)),
            pl.BlockSpec((1, DIM), lambda i: (0, 0)),
        ],
        out_specs=pl.BlockSpec((RB, DIM), lambda i: (i, 0)),
        out_shape=jax.ShapeDtypeStruct((N, DIM), jnp.float32),
    )(xh, x, mean, bv, dis, acc2, lsc, lbi, lw, lb, w1, b1, w2, b2)


def kernel(x, edge_index, W_gat, att_src, att_dst, bias_gat, lap_scale,
           lap_bias, ln2_w, ln2_b, W1, b1, W2, b2):
    src = edge_index[0].astype(jnp.int32)
    dst = edge_index[1].astype(jnp.int32)
    src1 = src.reshape(NW, NCH1, C1)
    dst1 = dst.reshape(NW, NCH1, C1)
    src2 = src.reshape(NW, NCH2, C2)
    dst2 = dst.reshape(NW, NCH2, C2)

    lanes = jnp.arange(DIM, dtype=jnp.int32)
    hsel = (lanes[:, None] // DH) == (jnp.arange(W16, dtype=jnp.int32) % HEADS)[None, :]
    as_w = jnp.where(hsel, att_src.reshape(DIM)[:, None], 0.0)
    ad_w = jnp.where(hsel, att_dst.reshape(DIM)[:, None], 0.0)
    sel8 = (lanes[:, None] // DH) == jnp.arange(HEADS, dtype=jnp.int32)[None, :]
    r_exp = sel8.T.astype(jnp.float32)

    h, as_tab, ad_tab, maxs, maxd = _tca(x, W_gat, as_w, ad_w)
    bnd = _lrelu(maxs + maxd)
    bnd16 = bnd.reshape(W16)

    z128 = jnp.zeros((RPT, DIM), jnp.float32)
    z16 = jnp.zeros((RPT, W16), jnp.float32)
    z8 = jnp.zeros((RPT, HEADS), jnp.float32)
    ones1 = jnp.ones((C1, HEADS), jnp.float32)

    degp = _sc0(src1, z8, ones1)
    nump, denp = _sc1(src1, dst1, as_tab, ad_tab, h, bnd16, z128, z16)

    xh, t1, dis, m, msq = _tcb(nump, denp, degp, h, as_tab, ad_tab, bnd,
                               r_exp, bias_gat.reshape(1, DIM))

    acc1 = _sc2(src2, dst2, t1, z16)
    t2, mean, bv = _tcb2(acc1, dis, m)
    acc2 = _sc2(src2, dst2, t2, z16)

    return _tcc(xh, x, mean, bv, dis, acc2,
                lap_scale.reshape(1, DIM), lap_bias.reshape(1, DIM),
                ln2_w.reshape(1, DIM), ln2_b.reshape(1, DIM),
                W1, b1.reshape(1, 2 * DIM), W2, b2.reshape(1, DIM))
